# baseline (speedup 1.0000x reference)
import functools

import jax
import jax.numpy as jnp
from jax import lax
from jax.experimental import pallas as pl
from jax.experimental.pallas import tpu as pltpu
from jax.experimental.pallas import tpu_sc as plsc

_NC = 2
_NS = 16
_NW = _NC * _NS
_L = 16


@functools.partial(jax.jit, static_argnums=(3, 4))
def _sc_add_gather(x, idx, tab, n_rows, d):
    rows_per_w = n_rows // _NW
    C = 64
    n_chunks = rows_per_w // C
    mesh = plsc.VectorSubcoreMesh(core_axis_name="c", subcore_axis_name="s")

    @functools.partial(
        pl.kernel,
        out_type=jax.ShapeDtypeStruct((n_rows, d), jnp.float32),
        mesh=mesh,
        scratch_types=[
            pltpu.VMEM((C,), jnp.int32),
            pltpu.VMEM((C, d), jnp.float32),
            pltpu.VMEM((C, d), jnp.float32),
            pltpu.SemaphoreType.DMA,
            pltpu.SemaphoreType.DMA,
        ],
    )
    def k(x_hbm, idx_hbm, tab_hbm, out_hbm, idx_v, in_v, row_v, sem_in, sem_tab):
        wid = lax.axis_index("s") * _NC + lax.axis_index("c")
        base = wid * rows_per_w

        def chunk(ci, _):
            rb = base + ci * C
            cp_in = pltpu.async_copy(x_hbm.at[pl.ds(rb, C)], in_v, sem_in)
            pltpu.sync_copy(idx_hbm.at[pl.ds(rb, C)], idx_v)
            cp_tab = pltpu.async_copy(tab_hbm.at[idx_v], row_v, sem_tab)
            cp_in.wait()
            cp_tab.wait()

            @plsc.parallel_loop(0, C, 1, unroll=2)
            def _radd(r):
                for j in range(d // _L):
                    plsc.addupdate(
                        in_v.at[r, pl.ds(j * _L, _L)],
                        row_v[r, pl.ds(j * _L, _L)],
                    )
            pltpu.sync_copy(in_v, out_hbm.at[pl.ds(rb, C)])
            return 0

        lax.fori_loop(0, n_chunks, chunk, 0)

    return k(x, idx, tab)


def kernel(inputs, spatial_pos_grid_size, inputs_positions, pos_embedding):
    bs, T, d = inputs.shape
    n_rows = bs * T
    x = inputs.reshape(n_rows, d)
    idx = inputs_positions.reshape(n_rows).astype(jnp.int32)
    tab = pos_embedding[0]
    out = _sc_add_gather(x, idx, tab, n_rows, d)
    return out.reshape(bs, T, d)


# speedup vs baseline: 1.4049x; 1.4049x over previous
import functools

import jax
import jax.numpy as jnp
from jax import lax
from jax.experimental import pallas as pl
from jax.experimental.pallas import tpu as pltpu
from jax.experimental.pallas import tpu_sc as plsc

_NC = 2
_NS = 16
_NW = _NC * _NS
_L = 16


@functools.partial(jax.jit, static_argnums=(3, 4))
def _sc_add_gather(x, idx, tab, n_rows, d):
    rows_per_w = n_rows // _NW
    C = 32
    NBUF = 2
    n_chunks = rows_per_w // C
    n_grp = n_chunks // NBUF
    mesh = plsc.VectorSubcoreMesh(core_axis_name="c", subcore_axis_name="s")

    @functools.partial(
        pl.kernel,
        out_type=jax.ShapeDtypeStruct((n_rows, d), jnp.float32),
        mesh=mesh,
        scratch_types=(
            [pltpu.VMEM((rows_per_w,), jnp.int32)]
            + [pltpu.VMEM((C, d), jnp.float32) for _ in range(2 * NBUF)]
            + [pltpu.SemaphoreType.DMA for _ in range(3 * NBUF)]
        ),
    )
    def k(x_hbm, idx_hbm, tab_hbm, out_hbm, idx_v, *bufs_and_sems):
        in_v = bufs_and_sems[:NBUF]
        row_v = bufs_and_sems[NBUF:2 * NBUF]
        sem_in = bufs_and_sems[2 * NBUF:3 * NBUF]
        sem_tab = bufs_and_sems[3 * NBUF:4 * NBUF]
        sem_out = bufs_and_sems[4 * NBUF:5 * NBUF]

        wid = lax.axis_index("s") * _NC + lax.axis_index("c")
        base = wid * rows_per_w

        pltpu.sync_copy(idx_hbm.at[pl.ds(base, rows_per_w)], idx_v)

        def issue(ci, b):
            rb = base + ci * C
            pltpu.async_copy(x_hbm.at[pl.ds(rb, C)], in_v[b], sem_in[b])
            pltpu.async_copy(
                tab_hbm.at[idx_v.at[pl.ds(ci * C, C)]], row_v[b], sem_tab[b]
            )

        def add_and_store(ci, b):
            pltpu.make_async_copy(x_hbm.at[pl.ds(0, C)], in_v[b], sem_in[b]).wait()
            pltpu.make_async_copy(
                tab_hbm.at[idx_v.at[pl.ds(0, C)]], row_v[b], sem_tab[b]
            ).wait()

            @plsc.parallel_loop(0, C, 1, unroll=2)
            def _radd(r):
                for j in range(d // _L):
                    plsc.addupdate(
                        in_v[b].at[r, pl.ds(j * _L, _L)],
                        row_v[b][r, pl.ds(j * _L, _L)],
                    )

            pltpu.async_copy(in_v[b], out_hbm.at[pl.ds(base + ci * C, C)], sem_out[b])

        for b in range(NBUF):
            issue(b, b)

        def grp(g, _):
            for b in range(NBUF):
                ci = g * NBUF + b
                add_and_store(ci, b)

                @pl.when(g < n_grp - 1)
                def _refill():
                    pltpu.make_async_copy(
                        in_v[b], out_hbm.at[pl.ds(0, C)], sem_out[b]
                    ).wait()
                    issue(ci + NBUF, b)

            return 0

        lax.fori_loop(0, n_grp, grp, 0)

        for b in range(NBUF):
            pltpu.make_async_copy(in_v[b], out_hbm.at[pl.ds(0, C)], sem_out[b]).wait()

    return k(x, idx, tab)


def kernel(inputs, spatial_pos_grid_size, inputs_positions, pos_embedding):
    bs, T, d = inputs.shape
    n_rows = bs * T
    x = inputs.reshape(n_rows, d)
    idx = inputs_positions.reshape(n_rows).astype(jnp.int32)
    tab = pos_embedding[0]
    out = _sc_add_gather(x, idx, tab, n_rows, d)
    return out.reshape(bs, T, d)


# speedup vs baseline: 1.4192x; 1.0102x over previous
import functools

import jax
import jax.numpy as jnp
from jax import lax
from jax.experimental import pallas as pl
from jax.experimental.pallas import tpu as pltpu
from jax.experimental.pallas import tpu_sc as plsc

_NC = 2
_NS = 16
_NW = _NC * _NS
_L = 16


@functools.partial(jax.jit, static_argnums=(3, 4))
def _sc_add_gather(x, idx, tab, n_rows, d):
    rows_per_w = n_rows // _NW
    C = 24
    NBUF = 3
    n_chunks = rows_per_w // C
    n_grp = n_chunks // NBUF
    mesh = plsc.VectorSubcoreMesh(core_axis_name="c", subcore_axis_name="s")

    @functools.partial(
        pl.kernel,
        out_type=jax.ShapeDtypeStruct((n_rows, d), jnp.float32),
        mesh=mesh,
        scratch_types=(
            [pltpu.VMEM((rows_per_w,), jnp.int32)]
            + [pltpu.VMEM((C, d), jnp.float32) for _ in range(2 * NBUF)]
            + [pltpu.SemaphoreType.DMA for _ in range(3 * NBUF)]
        ),
    )
    def k(x_hbm, idx_hbm, tab_hbm, out_hbm, idx_v, *bufs_and_sems):
        in_v = bufs_and_sems[:NBUF]
        row_v = bufs_and_sems[NBUF:2 * NBUF]
        sem_in = bufs_and_sems[2 * NBUF:3 * NBUF]
        sem_tab = bufs_and_sems[3 * NBUF:4 * NBUF]
        sem_out = bufs_and_sems[4 * NBUF:5 * NBUF]

        wid = lax.axis_index("s") * _NC + lax.axis_index("c")
        base = wid * rows_per_w

        pltpu.sync_copy(idx_hbm.at[pl.ds(base, rows_per_w)], idx_v)

        def issue(ci, b):
            rb = base + ci * C
            pltpu.async_copy(x_hbm.at[pl.ds(rb, C)], in_v[b], sem_in[b])
            pltpu.async_copy(
                tab_hbm.at[idx_v.at[pl.ds(ci * C, C)]], row_v[b], sem_tab[b]
            )

        def add_and_store(ci, b):
            pltpu.make_async_copy(x_hbm.at[pl.ds(0, C)], in_v[b], sem_in[b]).wait()
            pltpu.make_async_copy(
                tab_hbm.at[idx_v.at[pl.ds(0, C)]], row_v[b], sem_tab[b]
            ).wait()

            @plsc.parallel_loop(0, C, 1, unroll=2)
            def _radd(r):
                for j in range(d // _L):
                    plsc.addupdate(
                        in_v[b].at[r, pl.ds(j * _L, _L)],
                        row_v[b][r, pl.ds(j * _L, _L)],
                    )

            pltpu.async_copy(in_v[b], out_hbm.at[pl.ds(base + ci * C, C)], sem_out[b])

        for b in range(NBUF):
            issue(b, b)

        def grp(g, _):
            for b in range(NBUF):
                ci = g * NBUF + b
                add_and_store(ci, b)

                @pl.when(g < n_grp - 1)
                def _refill():
                    pltpu.make_async_copy(
                        in_v[b], out_hbm.at[pl.ds(0, C)], sem_out[b]
                    ).wait()
                    issue(ci + NBUF, b)

            return 0

        lax.fori_loop(0, n_grp, grp, 0)

        for b in range(NBUF):
            pltpu.make_async_copy(in_v[b], out_hbm.at[pl.ds(0, C)], sem_out[b]).wait()

    return k(x, idx, tab)


def kernel(inputs, spatial_pos_grid_size, inputs_positions, pos_embedding):
    bs, T, d = inputs.shape
    n_rows = bs * T
    x = inputs.reshape(n_rows, d)
    idx = inputs_positions.reshape(n_rows).astype(jnp.int32)
    tab = pos_embedding[0]
    out = _sc_add_gather(x, idx, tab, n_rows, d)
    return out.reshape(bs, T, d)


# speedup vs baseline: 1.4349x; 1.0111x over previous
import functools

import jax
import jax.numpy as jnp
from jax import lax
from jax.experimental import pallas as pl
from jax.experimental.pallas import tpu as pltpu
from jax.experimental.pallas import tpu_sc as plsc

_NC = 2
_NS = 16
_NW = _NC * _NS
_L = 16


@functools.partial(jax.jit, static_argnums=(3, 4))
def _sc_add_gather(x, idx, tab, n_rows, d):
    rows_per_w = n_rows // _NW
    C = 16
    NBUF = 3
    n_chunks = rows_per_w // C
    n_grp = n_chunks // NBUF
    mesh = plsc.VectorSubcoreMesh(core_axis_name="c", subcore_axis_name="s")

    @functools.partial(
        pl.kernel,
        out_type=jax.ShapeDtypeStruct((n_rows, d), jnp.float32),
        mesh=mesh,
        scratch_types=(
            [pltpu.VMEM((rows_per_w,), jnp.int32)]
            + [pltpu.VMEM((C, d), jnp.float32) for _ in range(3 * NBUF)]
            + [pltpu.SemaphoreType.DMA for _ in range(3 * NBUF)]
        ),
    )
    def k(x_hbm, idx_hbm, tab_hbm, out_hbm, idx_v, *bufs_and_sems):
        in_v = bufs_and_sems[:NBUF]
        row_v = bufs_and_sems[NBUF:2 * NBUF]
        out_v = bufs_and_sems[2 * NBUF:3 * NBUF]
        sem_in = bufs_and_sems[3 * NBUF:4 * NBUF]
        sem_tab = bufs_and_sems[4 * NBUF:5 * NBUF]
        sem_out = bufs_and_sems[5 * NBUF:6 * NBUF]

        wid = lax.axis_index("s") * _NC + lax.axis_index("c")
        base = wid * rows_per_w

        pltpu.sync_copy(idx_hbm.at[pl.ds(base, rows_per_w)], idx_v)

        def issue(ci, b):
            rb = base + ci * C
            pltpu.async_copy(x_hbm.at[pl.ds(rb, C)], in_v[b], sem_in[b])
            pltpu.async_copy(
                tab_hbm.at[idx_v.at[pl.ds(ci * C, C)]], row_v[b], sem_tab[b]
            )

        for b in range(NBUF):
            issue(b, b)

        def grp(g, _):
            for b in range(NBUF):
                ci = g * NBUF + b
                pltpu.make_async_copy(x_hbm.at[pl.ds(0, C)], in_v[b], sem_in[b]).wait()
                pltpu.make_async_copy(
                    tab_hbm.at[idx_v.at[pl.ds(0, C)]], row_v[b], sem_tab[b]
                ).wait()

                @pl.when(g > 0)
                def _drain():
                    pltpu.make_async_copy(
                        out_v[b], out_hbm.at[pl.ds(0, C)], sem_out[b]
                    ).wait()

                @plsc.parallel_loop(0, C, 1, unroll=2)
                def _radd(r):
                    for j in range(d // _L):
                        out_v[b][r, pl.ds(j * _L, _L)] = (
                            in_v[b][r, pl.ds(j * _L, _L)]
                            + row_v[b][r, pl.ds(j * _L, _L)]
                        )

                pltpu.async_copy(
                    out_v[b], out_hbm.at[pl.ds(base + ci * C, C)], sem_out[b]
                )

                @pl.when(g < n_grp - 1)
                def _refill():
                    issue(ci + NBUF, b)

            return 0

        lax.fori_loop(0, n_grp, grp, 0)

        for b in range(NBUF):
            pltpu.make_async_copy(out_v[b], out_hbm.at[pl.ds(0, C)], sem_out[b]).wait()

    return k(x, idx, tab)


def kernel(inputs, spatial_pos_grid_size, inputs_positions, pos_embedding):
    bs, T, d = inputs.shape
    n_rows = bs * T
    x = inputs.reshape(n_rows, d)
    idx = inputs_positions.reshape(n_rows).astype(jnp.int32)
    tab = pos_embedding[0]
    out = _sc_add_gather(x, idx, tab, n_rows, d)
    return out.reshape(bs, T, d)
